# baseline (speedup 1.0000x reference)
import functools

import jax
import jax.numpy as jnp
from jax import lax
from jax.experimental import pallas as pl
from jax.experimental.pallas import tpu as pltpu
from jax.experimental.pallas import tpu_sc as plsc

_NUM_WORKERS = 32
_LANES = 128


def _make_emb_kernel(bsz, seq, d):
    mesh = plsc.VectorSubcoreMesh(core_axis_name="c", subcore_axis_name="s")
    ndg = d // 8
    nblk = _LANES // 16

    @functools.partial(
        pl.kernel,
        mesh=mesh,
        out_type=jax.ShapeDtypeStruct((seq, ndg, _NUM_WORKERS, 8, _LANES),
                                      jnp.float32),
        scratch_types=[
            pltpu.VMEM((seq, _LANES), jnp.int32),
            pltpu.VMEM((3, _LANES, d), jnp.float32),
            pltpu.VMEM((2, d, _LANES), jnp.float32),
            pltpu.SemaphoreType.DMA((3,)),
            pltpu.SemaphoreType.DMA((2,)),
        ],
        compiler_params=pltpu.CompilerParams(
            use_tc_tiling_on_sc=False, needs_layout_passes=False),
    )
    def emb(x_hbm, tab_hbm, out_hbm, idx_v, rows_v, tbuf_v, gsem, osem):
        wid = lax.axis_index("s") * 2 + lax.axis_index("c")
        lane0 = wid * _LANES
        pltpu.sync_copy(x_hbm.at[:, pl.ds(lane0, _LANES)], idx_v)

        iot = lax.iota(jnp.int32, 16)
        iotbs = [iot + blk * 16 for blk in range(nblk)]

        def fire_gather(s, rb):
            pltpu.async_copy(tab_hbm.at[idx_v.at[s]], rows_v.at[rb],
                             gsem.at[rb])

        def wait_gather(rb):
            pltpu.make_async_copy(
                tab_hbm.at[idx_v.at[0]], rows_v.at[rb], gsem.at[rb]).wait()

        def fire_out(s, tb):
            for dg in range(ndg):
                pltpu.async_copy(
                    tbuf_v.at[tb, pl.ds(dg * 8, 8)],
                    out_hbm.at[s, dg, wid], osem.at[tb])

        def wait_out(tb):
            for dg in range(ndg):
                pltpu.make_async_copy(
                    tbuf_v.at[tb, pl.ds(dg * 8, 8)],
                    out_hbm.at[0, dg, wid], osem.at[tb]).wait()

        def transpose_chunk(rb, tb):
            for dg4 in range(d // 4):
                vals = []
                for du in range(4):
                    col = lax.broadcast(dg4 * 4 + du, (16,))
                    for blk in range(nblk):
                        vals.append(plsc.load_gather(
                            rows_v.at[rb], [iotbs[blk], col]))
                for du in range(4):
                    for blk in range(nblk):
                        tbuf_v[tb, dg4 * 4 + du, pl.ds(blk * 16, 16)] = (
                            vals[du * nblk + blk])

        def step(s, rb, tb, fire_ahead=True, wait_o=True):
            if wait_o:
                wait_out(tb)
            if fire_ahead:
                fire_gather(s + 2, (rb + 2) % 3)
            wait_gather(rb)
            transpose_chunk(rb, tb)
            fire_out(s, tb)

        fire_gather(0, 0)
        fire_gather(1, 1)
        step(0, 0, 0, wait_o=False)
        step(1, 1, 1, wait_o=False)

        n_groups = (seq - 8) // 6
        assert (seq - 8) % 6 == 0

        def group(k, _):
            g = 2 + 6 * k
            for u in range(6):
                step(g + u, (2 + u) % 3, u % 2)
            return 0

        lax.fori_loop(0, n_groups, group, 0)

        for s in range(seq - 6, seq - 2):
            step(s, s % 3, s % 2)
        for s in range(seq - 2, seq):
            step(s, s % 3, s % 2, fire_ahead=False)
        for tb in range(2):
            wait_out(tb)

    return emb


def kernel(x, table):
    bsz, seq = x.shape
    v, d = table.shape
    assert bsz == _NUM_WORKERS * _LANES and d % 8 == 0
    xt = x.T.astype(jnp.int32)
    out5 = _make_emb_kernel(bsz, seq, d)(xt, table)
    out3 = out5.transpose(0, 1, 3, 2, 4).reshape(seq, d, bsz)
    return out3.transpose(2, 0, 1)


# speedup vs baseline: 1.0154x; 1.0154x over previous
import functools

import jax
import jax.numpy as jnp
from jax import lax
from jax.experimental import pallas as pl
from jax.experimental.pallas import tpu as pltpu
from jax.experimental.pallas import tpu_sc as plsc

_NUM_WORKERS = 32
_NBUF = 4
_LOOKAHEAD = 2


def _make_emb_kernel(bsz, seq, d):
    mesh = plsc.VectorSubcoreMesh(core_axis_name="c", subcore_axis_name="s")
    n_chunks = bsz // _NUM_WORKERS
    slack = _NBUF - _LOOKAHEAD
    assert slack >= 1 and n_chunks % _NBUF == 0 and n_chunks >= 2 * _NBUF

    @functools.partial(
        pl.kernel,
        mesh=mesh,
        out_type=jax.ShapeDtypeStruct((bsz, seq, d), jnp.float32),
        scratch_types=[
            pltpu.VMEM((n_chunks, seq), jnp.int32),
            pltpu.VMEM((_NBUF, seq, d), jnp.float32),
            pltpu.SemaphoreType.DMA((_NBUF,)),
            pltpu.SemaphoreType.DMA((_NBUF,)),
        ],
        compiler_params=pltpu.CompilerParams(use_tc_tiling_on_sc=False),
    )
    def emb(x_hbm, tab_hbm, out_hbm, idx_v, rows_v, gsem, osem):
        wid = lax.axis_index("s") * 2 + lax.axis_index("c")
        base = wid * n_chunks
        pltpu.sync_copy(x_hbm.at[wid], idx_v)

        def fire_gather(j, b):
            pltpu.async_copy(tab_hbm.at[idx_v.at[j]], rows_v.at[b], gsem.at[b])

        def wait_gather(b):
            pltpu.make_async_copy(
                tab_hbm.at[idx_v.at[0]], rows_v.at[b], gsem.at[b]).wait()

        def fire_out(j, b):
            pltpu.async_copy(rows_v.at[b], out_hbm.at[base + j], osem.at[b])

        def wait_out(b):
            pltpu.make_async_copy(
                rows_v.at[b], out_hbm.at[base], osem.at[b]).wait()

        for j in range(_LOOKAHEAD):
            fire_gather(j, j % _NBUF)
        for j in range(slack):
            wait_gather(j % _NBUF)
            fire_out(j, j % _NBUF)
            fire_gather(j + _LOOKAHEAD, (j + _LOOKAHEAD) % _NBUF)

        def group(k, _):
            g = slack + _NBUF * k
            for u in range(_NBUF):
                j = g + u
                b = (slack + u) % _NBUF
                bb = (slack + u + _LOOKAHEAD) % _NBUF
                wait_out(bb)
                fire_gather(j + _LOOKAHEAD, bb)
                wait_gather(b)
                fire_out(j, b)
            return 0

        n_main = n_chunks - _LOOKAHEAD - slack
        assert n_main % _NBUF == 0
        lax.fori_loop(0, n_main // _NBUF, group, 0)

        for j in range(n_chunks - _LOOKAHEAD, n_chunks):
            wait_gather(j % _NBUF)
            fire_out(j, j % _NBUF)
        for b in range(_NBUF):
            wait_out(b)

    return emb


def kernel(x, table):
    bsz, seq = x.shape
    v, d = table.shape
    assert bsz % _NUM_WORKERS == 0
    x3 = x.reshape(_NUM_WORKERS, bsz // _NUM_WORKERS, seq).astype(jnp.int32)
    return _make_emb_kernel(bsz, seq, d)(x3, table)
